# baseline (speedup 1.0000x reference)
import functools

import jax
import jax.numpy as jnp
from jax import lax
from jax.experimental import pallas as pl
from jax.experimental.pallas import tpu as pltpu
from jax.experimental.pallas import tpu_sc as plsc

BATCH = 16384
N_FIELDS = 26
D_EMB = 64
F_PAD = 32
D_PAD = 2 * D_EMB
N_ROWS = BATCH * F_PAD

_NC = 2
_NS = 16
_NW = _NC * _NS
_ROWS_PER_W = N_ROWS // _NW
_CROWS = 256
_N_CHUNKS = _ROWS_PER_W // _CROWS
_HALF = _N_CHUNKS // 2

_mesh = plsc.VectorSubcoreMesh(core_axis_name="c", subcore_axis_name="s")


@functools.partial(
    pl.kernel,
    mesh=_mesh,
    out_type=jax.ShapeDtypeStruct((N_ROWS, D_PAD), jnp.float32),
    scratch_types=[
        pltpu.VMEM((_CROWS,), jnp.int32),
        pltpu.VMEM((_CROWS,), jnp.int32),
        pltpu.VMEM((_CROWS, D_PAD), jnp.float32),
        pltpu.VMEM((_CROWS, D_PAD), jnp.float32),
        pltpu.SemaphoreType.DMA,
        pltpu.SemaphoreType.DMA,
        pltpu.SemaphoreType.DMA,
        pltpu.SemaphoreType.DMA,
    ],
    compiler_params=pltpu.CompilerParams(use_tc_tiling_on_sc=True),
)
def _gather_sc(idx_hbm, table_hbm, out_hbm, idx0, idx1, rows0, rows1,
               gsem0, gsem1, osem0, osem1):
    wid = lax.axis_index("s") * _NC + lax.axis_index("c")
    base = wid * _ROWS_PER_W

    idxb = (idx0, idx1)
    rows = (rows0, rows1)
    gsem = (gsem0, gsem1)
    osem = (osem0, osem1)

    def stage_idx(c, b):
        pltpu.sync_copy(idx_hbm.at[pl.ds(base + c * _CROWS, _CROWS)], idxb[b])

    def gather_start(b):
        pltpu.async_copy(table_hbm.at[idxb[b]], rows[b], gsem[b])

    def gather_wait(b):
        pltpu.make_async_copy(table_hbm.at[idxb[b]], rows[b], gsem[b]).wait()

    def out_start(c, b):
        pltpu.async_copy(
            rows[b], out_hbm.at[pl.ds(base + c * _CROWS, _CROWS)], osem[b])

    def out_wait(b):
        pltpu.make_async_copy(
            rows[b], out_hbm.at[pl.ds(base, _CROWS)], osem[b]).wait()

    stage_idx(0, 0)
    gather_start(0)
    stage_idx(1, 1)
    gather_start(1)

    @pl.loop(0, _HALF - 1)
    def _body(i):
        c0 = 2 * i
        gather_wait(0)
        out_start(c0, 0)
        gather_wait(1)
        out_start(c0 + 1, 1)
        out_wait(0)
        stage_idx(c0 + 2, 0)
        gather_start(0)
        out_wait(1)
        stage_idx(c0 + 3, 1)
        gather_start(1)

    gather_wait(0)
    out_start(_N_CHUNKS - 2, 0)
    gather_wait(1)
    out_start(_N_CHUNKS - 1, 1)
    out_wait(0)
    out_wait(1)


def kernel(idx, table):
    idx32 = jnp.pad(idx.astype(jnp.int32), ((0, 0), (0, F_PAD - N_FIELDS)))
    table128 = jnp.pad(table, ((0, 0), (0, D_PAD - D_EMB)))
    out = _gather_sc(idx32.reshape(-1), table128)
    return out[:, :D_EMB].reshape(BATCH, F_PAD, D_EMB)[:, :N_FIELDS, :]


# speedup vs baseline: 1.0004x; 1.0004x over previous
import functools

import jax
import jax.numpy as jnp
from jax import lax
from jax.experimental import pallas as pl
from jax.experimental.pallas import tpu as pltpu
from jax.experimental.pallas import tpu_sc as plsc

BATCH = 16384
N_FIELDS = 26
D_EMB = 64
F_PAD = 32
D_PAD = 2 * D_EMB
N_ROWS = BATCH * F_PAD

_NC = 2
_NS = 16
_NW = _NC * _NS
_ROWS_PER_W = N_ROWS // _NW
_CROWS = 496
_SIZES = [_CROWS] * (_ROWS_PER_W // _CROWS) + [_ROWS_PER_W % _CROWS]
_OFFS = [sum(_SIZES[:i]) for i in range(len(_SIZES))]
_N_CHUNKS = len(_SIZES)

_mesh = plsc.VectorSubcoreMesh(core_axis_name="c", subcore_axis_name="s")


@functools.partial(
    pl.kernel,
    mesh=_mesh,
    out_type=jax.ShapeDtypeStruct((N_ROWS, D_PAD), jnp.float32),
    scratch_types=[
        pltpu.VMEM((_CROWS,), jnp.int32),
        pltpu.VMEM((_CROWS,), jnp.int32),
        pltpu.VMEM((_CROWS, D_PAD), jnp.float32),
        pltpu.VMEM((_CROWS, D_PAD), jnp.float32),
        pltpu.SemaphoreType.DMA,
        pltpu.SemaphoreType.DMA,
        pltpu.SemaphoreType.DMA,
        pltpu.SemaphoreType.DMA,
        pltpu.SemaphoreType.DMA,
        pltpu.SemaphoreType.DMA,
    ],
    compiler_params=pltpu.CompilerParams(use_tc_tiling_on_sc=True),
)
def _gather_sc(idx_hbm, table_hbm, out_hbm, idx0, idx1, rows0, rows1,
               isem0, isem1, gsem0, gsem1, osem0, osem1):
    wid = lax.axis_index("s") * _NC + lax.axis_index("c")
    base = wid * _ROWS_PER_W

    idxb = (idx0, idx1)
    rows = (rows0, rows1)
    isem = (isem0, isem1)
    gsem = (gsem0, gsem1)
    osem = (osem0, osem1)

    def idx_start(i):
        b = i % 2
        return pltpu.async_copy(
            idx_hbm.at[pl.ds(base + _OFFS[i], _SIZES[i])],
            idxb[b].at[pl.ds(0, _SIZES[i])], isem[b])

    def gather_start(i):
        b = i % 2
        return pltpu.async_copy(
            table_hbm.at[idxb[b].at[pl.ds(0, _SIZES[i])]],
            rows[b].at[pl.ds(0, _SIZES[i])], gsem[b])

    def out_start(i):
        b = i % 2
        return pltpu.async_copy(
            rows[b].at[pl.ds(0, _SIZES[i])],
            out_hbm.at[pl.ds(base + _OFFS[i], _SIZES[i])], osem[b])

    g = [None] * _N_CHUNKS
    o = [None] * _N_CHUNKS
    idx_start(0).wait()
    g[0] = gather_start(0)
    idx_start(1).wait()
    g[1] = gather_start(1)
    for i in range(_N_CHUNKS):
        g[i].wait()
        o[i] = out_start(i)
        if i + 2 < _N_CHUNKS:
            ic = idx_start(i + 2)
            o[i].wait()
            ic.wait()
            g[i + 2] = gather_start(i + 2)
    o[_N_CHUNKS - 2].wait()
    o[_N_CHUNKS - 1].wait()


def kernel(idx, table):
    idx32 = jnp.pad(idx.astype(jnp.int32), ((0, 0), (0, F_PAD - N_FIELDS)))
    table128 = jnp.pad(table, ((0, 0), (0, D_PAD - D_EMB)))
    out = _gather_sc(idx32.reshape(-1), table128)
    return out[:, :D_EMB].reshape(BATCH, F_PAD, D_EMB)[:, :N_FIELDS, :]


# speedup vs baseline: 5.9390x; 5.9365x over previous
import functools

import jax
import jax.numpy as jnp
from jax import lax
from jax.experimental import pallas as pl
from jax.experimental.pallas import tpu as pltpu
from jax.experimental.pallas import tpu_sc as plsc

BATCH = 16384
N_FIELDS = 26
D_EMB = 64
F_PAD = 32
D_PAD = 2 * D_EMB
N_ROWS = BATCH * F_PAD

_NC = 2
_NS = 16
_NW = _NC * _NS
_ROWS_PER_W = N_ROWS // _NW
_CROWS = 496
_SIZES = [_CROWS] * (_ROWS_PER_W // _CROWS) + [_ROWS_PER_W % _CROWS]
_OFFS = [sum(_SIZES[:i]) for i in range(len(_SIZES))]
_N_CHUNKS = len(_SIZES)

_mesh = plsc.VectorSubcoreMesh(core_axis_name="c", subcore_axis_name="s")


@functools.partial(
    pl.kernel,
    mesh=_mesh,
    out_type=jax.ShapeDtypeStruct((N_ROWS, D_PAD), jnp.float32),
    scratch_types=[
        pltpu.VMEM((_CROWS,), jnp.int32),
        pltpu.VMEM((_CROWS,), jnp.int32),
        pltpu.VMEM((_CROWS, D_PAD), jnp.float32),
        pltpu.VMEM((_CROWS, D_PAD), jnp.float32),
        pltpu.SemaphoreType.DMA,
        pltpu.SemaphoreType.DMA,
        pltpu.SemaphoreType.DMA,
        pltpu.SemaphoreType.DMA,
        pltpu.SemaphoreType.DMA,
        pltpu.SemaphoreType.DMA,
    ],
    compiler_params=pltpu.CompilerParams(use_tc_tiling_on_sc=True),
)
def _gather_sc(idx_hbm, table_hbm, out_hbm, idx0, idx1, rows0, rows1,
               isem0, isem1, gsem0, gsem1, osem0, osem1):
    wid = lax.axis_index("s") * _NC + lax.axis_index("c")
    base = wid * _ROWS_PER_W

    idxb = (idx0, idx1)
    rows = (rows0, rows1)
    isem = (isem0, isem1)
    gsem = (gsem0, gsem1)
    osem = (osem0, osem1)

    def idx_start(i):
        b = i % 2
        return pltpu.async_copy(
            idx_hbm.at[pl.ds(base + _OFFS[i], _SIZES[i])],
            idxb[b].at[pl.ds(0, _SIZES[i])], isem[b])

    def gather_start(i):
        b = i % 2
        return pltpu.async_copy(
            table_hbm.at[idxb[b].at[pl.ds(0, _SIZES[i])]],
            rows[b].at[pl.ds(0, _SIZES[i])], gsem[b])

    def out_start(i):
        b = i % 2
        return pltpu.async_copy(
            rows[b].at[pl.ds(0, _SIZES[i])],
            out_hbm.at[pl.ds(base + _OFFS[i], _SIZES[i])], osem[b])

    g = [None] * _N_CHUNKS
    o = [None] * _N_CHUNKS
    idx_start(0).wait()
    g[0] = gather_start(0)
    idx_start(1).wait()
    g[1] = gather_start(1)
    for i in range(_N_CHUNKS):
        g[i].wait()
        o[i] = out_start(i)
        if i + 2 < _N_CHUNKS:
            ic = idx_start(i + 2)
            o[i].wait()
            ic.wait()
            g[i + 2] = gather_start(i + 2)
    o[_N_CHUNKS - 2].wait()
    o[_N_CHUNKS - 1].wait()


def kernel(idx, table):
    fill = (jnp.arange(BATCH * (F_PAD - N_FIELDS), dtype=jnp.int32)
            .reshape(BATCH, F_PAD - N_FIELDS) * 10) % 1000000
    idx32 = jnp.concatenate([idx.astype(jnp.int32), fill], axis=1)
    table128 = jnp.pad(table, ((0, 0), (0, D_PAD - D_EMB)))
    out = _gather_sc(idx32.reshape(-1), table128)
    return out[:, :D_EMB].reshape(BATCH, F_PAD, D_EMB)[:, :N_FIELDS, :]


# speedup vs baseline: 6.3546x; 1.0700x over previous
import functools

import jax
import jax.numpy as jnp
from jax import lax
from jax.experimental import pallas as pl
from jax.experimental.pallas import tpu as pltpu
from jax.experimental.pallas import tpu_sc as plsc

BATCH = 16384
N_FIELDS = 26
D_EMB = 64
D_PAD = 2 * D_EMB
N_ROWS = BATCH * N_FIELDS

_NC = 2
_NS = 16
_NW = _NC * _NS
_ROWS_PER_W = N_ROWS // _NW
_CROWS = 416
_N_CHUNKS = _ROWS_PER_W // _CROWS

_mesh = plsc.VectorSubcoreMesh(core_axis_name="c", subcore_axis_name="s")


@functools.partial(
    pl.kernel,
    mesh=_mesh,
    out_type=jax.ShapeDtypeStruct((N_ROWS, D_PAD), jnp.float32),
    scratch_types=[
        pltpu.VMEM((_ROWS_PER_W,), jnp.int32),
        pltpu.VMEM((_CROWS, D_PAD), jnp.float32),
        pltpu.VMEM((_CROWS, D_PAD), jnp.float32),
        pltpu.SemaphoreType.DMA,
        pltpu.SemaphoreType.DMA,
        pltpu.SemaphoreType.DMA,
        pltpu.SemaphoreType.DMA,
    ],
    compiler_params=pltpu.CompilerParams(use_tc_tiling_on_sc=True),
)
def _gather_sc(idx_hbm, table_hbm, out_hbm, idx_v, rows0, rows1,
               gsem0, gsem1, osem0, osem1):
    wid = lax.axis_index("s") * _NC + lax.axis_index("c")
    base = wid * _ROWS_PER_W
    pltpu.sync_copy(idx_hbm.at[pl.ds(base, _ROWS_PER_W)], idx_v)

    rows = (rows0, rows1)
    gsem = (gsem0, gsem1)
    osem = (osem0, osem1)

    def gather_start(i):
        b = i % 2
        return pltpu.async_copy(
            table_hbm.at[idx_v.at[pl.ds(i * _CROWS, _CROWS)]], rows[b], gsem[b])

    def out_start(i):
        b = i % 2
        return pltpu.async_copy(
            rows[b], out_hbm.at[pl.ds(base + i * _CROWS, _CROWS)], osem[b])

    g = [None] * _N_CHUNKS
    o = [None] * _N_CHUNKS
    g[0] = gather_start(0)
    g[1] = gather_start(1)
    for i in range(_N_CHUNKS):
        g[i].wait()
        o[i] = out_start(i)
        if i + 2 < _N_CHUNKS:
            o[i].wait()
            g[i + 2] = gather_start(i + 2)
    o[_N_CHUNKS - 2].wait()
    o[_N_CHUNKS - 1].wait()


def kernel(idx, table):
    idx_fmaj = idx.astype(jnp.int32).T.reshape(-1)
    table128 = jnp.pad(table, ((0, 0), (0, D_PAD - D_EMB)))
    out = _gather_sc(idx_fmaj, table128)
    return out[:, :D_EMB].reshape(N_FIELDS, BATCH, D_EMB).transpose(1, 0, 2)
